# baseline (speedup 1.0000x reference)
import functools

import jax
import jax.numpy as jnp
from jax import lax
from jax.experimental import pallas as pl
from jax.experimental.pallas import tpu as pltpu
from jax.experimental.pallas import tpu_sc as plsc

N_NODES = 10000
N_EDGES = 320000
F = 128
NC = 2
NS = 16
L = 16

N_PAD = 10240
ROWS_PER_TILE = N_PAD // NS
CHUNK = 128


def _sc_mesh():
    return plsc.VectorSubcoreMesh(
        core_axis_name="c", subcore_axis_name="s", num_cores=NC, num_subcores=NS)


HIST_PW = 10112


def _hist_body(idx_hbm, out_hbm, idxbuf, hist):
    c = lax.axis_index("c")
    s = lax.axis_index("s")
    w = c * NS + s
    ones = jnp.ones((L,), jnp.float32)
    zeros = jnp.zeros((L,), jnp.float32)
    for a in range(4):
        row = a * (NC * NS) + w
        pltpu.sync_copy(idx_hbm.at[row], idxbuf)

        def zstep(i, _):
            hist[pl.ds(i * L, L)] = zeros
            return 0

        lax.fori_loop(0, N_PAD // L, zstep, 0)

        def step(i, _):
            iv = idxbuf[pl.ds(i * L, L)]
            plsc.addupdate_scatter(hist, [iv], ones)
            return 0

        lax.fori_loop(0, HIST_PW // L, step, 0)
        pltpu.sync_copy(hist, out_hbm.at[row])


def _histograms(idx4):
    per_w = N_EDGES // (NC * NS)
    rows = idx4.reshape(4 * NC * NS, per_w)
    rows = jnp.pad(rows, ((0, 0), (0, HIST_PW - per_w)),
                   constant_values=N_PAD - 1)
    return pl.kernel(
        _hist_body,
        out_type=jax.ShapeDtypeStruct((4 * NC * NS, N_PAD), jnp.float32),
        mesh=_sc_mesh(),
        compiler_params=pltpu.CompilerParams(needs_layout_passes=False),
        scratch_types=[
            pltpu.VMEM((HIST_PW,), jnp.int32),
            pltpu.VMEM((N_PAD,), jnp.float32),
        ],
    )(rows)


SLAB = 16


def _agg_body(n_slabs, h_hbm, sd_hbm, zeros_hbm, out_hbm,
              sd_v, rows0, rows1, agg_sh, sem0, sem1):
    c = lax.axis_index("c")
    s = lax.axis_index("s")
    pltpu.sync_copy(zeros_hbm, rows0)
    for z in range(ROWS_PER_TILE // CHUNK):
        pltpu.sync_copy(rows0, agg_sh.at[pl.ds(s * ROWS_PER_TILE + z * CHUNK, CHUNK)])
    plsc.subcore_barrier()

    def outer(t, _):
        pltpu.sync_copy(sd_hbm.at[c].at[s].at[t], sd_v)

        def inner(p, _):
            g0 = pltpu.async_copy(h_hbm.at[sd_v.at[4 * p]], rows0, sem0)
            g1 = pltpu.async_copy(h_hbm.at[sd_v.at[4 * p + 2]], rows1, sem1)
            g0.wait()
            pltpu.sync_copy(rows0, agg_sh.at[sd_v.at[4 * p + 1]], add=True)
            g1.wait()
            pltpu.sync_copy(rows1, agg_sh.at[sd_v.at[4 * p + 3]], add=True)
            return 0

        lax.fori_loop(0, SLAB // 2, inner, 0)
        return 0

    lax.fori_loop(0, n_slabs, outer, 0)
    plsc.subcore_barrier()
    pltpu.sync_copy(agg_sh.at[pl.ds(s * ROWS_PER_TILE, ROWS_PER_TILE)],
                    out_hbm.at[c].at[pl.ds(s * ROWS_PER_TILE, ROWS_PER_TILE)])


def _aggregate(h, sd_slab):
    n_slabs = sd_slab.shape[2]
    zeros = jnp.zeros((CHUNK, F), jnp.float32)
    return pl.kernel(
        functools.partial(_agg_body, n_slabs),
        out_type=jax.ShapeDtypeStruct((2, N_PAD, F), jnp.float32),
        mesh=_sc_mesh(),
        compiler_params=pltpu.CompilerParams(needs_layout_passes=False),
        scratch_types=[
            pltpu.VMEM((2 * SLAB, CHUNK), jnp.int32),
            pltpu.VMEM((CHUNK, F), jnp.float32),
            pltpu.VMEM((CHUNK, F), jnp.float32),
            pltpu.VMEM_SHARED((N_PAD, F), jnp.float32),
            pltpu.SemaphoreType.DMA,
            pltpu.SemaphoreType.DMA,
        ],
    )(h, sd_slab, zeros)


def _slab(src, dst, n_chunks):
    def prep(idx, pad_value):
        per_tile = idx.shape[1] // NS
        idx = idx.reshape(2, NS, per_tile)
        pad = n_chunks * CHUNK - per_tile
        idx = jnp.pad(idx, ((0, 0), (0, 0), (0, pad)), constant_values=pad_value)
        return idx.reshape(2, NS, n_chunks, CHUNK)

    sd = jnp.stack([prep(src, 0), prep(dst, N_NODES)], axis=3)
    return sd.reshape(2, NS, n_chunks // SLAB, 2 * SLAB, CHUNK)


def _tc_a_body(x_ref, wcat_ref, deg_ref, hcat_ref, norms_ref):
    B = deg_ref.shape[1]
    degs = deg_ref[...].reshape(4, NC * NS, B).sum(axis=1)
    norms = lax.rsqrt(jnp.maximum(degs, 1.0))
    norms_ref[...] = norms
    y = jnp.dot(x_ref[...], wcat_ref[...], preferred_element_type=jnp.float32)
    hcat_ref[0] = y[:, :F] * norms[0][:, None]
    hcat_ref[1] = y[:, F:] * norms[2][:, None]


def _tc_a(x, wcat, deg_flat):
    B = 2048
    grid = (N_PAD // B,)
    return pl.pallas_call(
        _tc_a_body,
        grid=grid,
        in_specs=[
            pl.BlockSpec((B, F), lambda i: (i, 0)),
            pl.BlockSpec((F, 2 * F), lambda i: (0, 0)),
            pl.BlockSpec((4 * NC * NS, B), lambda i: (0, i)),
        ],
        out_specs=[
            pl.BlockSpec((2, B, F), lambda i: (0, i, 0)),
            pl.BlockSpec((4, B), lambda i: (0, i)),
        ],
        out_shape=[
            jax.ShapeDtypeStruct((2, N_PAD, F), jnp.float32),
            jax.ShapeDtypeStruct((4, N_PAD), jnp.float32),
        ],
    )(x, wcat, deg_flat)


def _tc_c_body(agg_ref, norms_ref, b1i_ref, b1b_ref, w2_ref, h2_ref):
    nd_i = norms_ref[1][:, None]
    nd_b = norms_ref[3][:, None]
    ns_i = norms_ref[0][:, None]
    h = (jnp.maximum(agg_ref[0] * nd_i + b1i_ref[...], 0.0)
         + jnp.maximum(agg_ref[1] * nd_b + b1b_ref[...], 0.0))
    h2_ref[...] = jnp.dot(h, w2_ref[...], preferred_element_type=jnp.float32) * ns_i


def _tc_c(agg, norms, b1i, b1b, w2):
    B = 2048
    return pl.pallas_call(
        _tc_c_body,
        grid=(N_PAD // B,),
        in_specs=[
            pl.BlockSpec((2, B, F), lambda i: (0, i, 0)),
            pl.BlockSpec((4, B), lambda i: (0, i)),
            pl.BlockSpec((1, F), lambda i: (0, 0)),
            pl.BlockSpec((1, F), lambda i: (0, 0)),
            pl.BlockSpec((F, F), lambda i: (0, 0)),
        ],
        out_specs=pl.BlockSpec((B, F), lambda i: (i, 0)),
        out_shape=jax.ShapeDtypeStruct((N_PAD, F), jnp.float32),
    )(agg, norms, b1i, b1b, w2)


def _tc_e_body(p_ref, norms_ref, b2_ref, out_ref):
    nd_i = norms_ref[1][:, None]
    out_ref[...] = (p_ref[0] + p_ref[1]) * nd_i + b2_ref[...]


def _tc_e(p, norms, b2):
    B = 2048
    return pl.pallas_call(
        _tc_e_body,
        grid=(N_PAD // B,),
        in_specs=[
            pl.BlockSpec((2, B, F), lambda i: (0, i, 0)),
            pl.BlockSpec((4, B), lambda i: (0, i)),
            pl.BlockSpec((1, F), lambda i: (0, 0)),
        ],
        out_specs=pl.BlockSpec((B, F), lambda i: (i, 0)),
        out_shape=jax.ShapeDtypeStruct((N_PAD, F), jnp.float32),
    )(p, norms, b2)


def kernel(x, edge_index_interacts, edge_index_behave, W1i, b1i, W1b, b1b, W2, b2):
    si = edge_index_interacts[0].astype(jnp.int32)
    di = edge_index_interacts[1].astype(jnp.int32)
    sb = edge_index_behave[0].astype(jnp.int32)
    db = edge_index_behave[1].astype(jnp.int32)

    idx4 = jnp.stack([si, di, sb, db])
    deg_flat = _histograms(idx4)

    x_pad = jnp.pad(x, ((0, N_PAD - N_NODES), (0, 0)))
    wcat = jnp.concatenate([W1i, W1b], axis=1)
    hcat, norms = _tc_a(x_pad, wcat, deg_flat)
    h_gather = hcat.reshape(2 * N_PAD, F)

    n_chunks1 = -(-(N_EDGES // NS) // CHUNK)
    n_chunks1 = -(-n_chunks1 // SLAB) * SLAB
    sd1 = _slab(jnp.stack([si, sb + N_PAD]), jnp.stack([di, db]), n_chunks1)
    agg1 = _aggregate(h_gather, sd1)

    h2 = _tc_c(agg1, norms, b1i.reshape(1, F), b1b.reshape(1, F), W2)

    n_chunks2 = -(-(N_EDGES // (2 * NS)) // CHUNK)
    n_chunks2 = -(-n_chunks2 // SLAB) * SLAB
    sd2 = _slab(si.reshape(2, N_EDGES // 2), di.reshape(2, N_EDGES // 2), n_chunks2)
    p2 = _aggregate(h2, sd2)

    return _tc_e(p2, norms, b2.reshape(1, F))[:N_NODES]


# speedup vs baseline: 1.0054x; 1.0054x over previous
import functools

import jax
import jax.numpy as jnp
from jax import lax
from jax.experimental import pallas as pl
from jax.experimental.pallas import tpu as pltpu
from jax.experimental.pallas import tpu_sc as plsc

N_NODES = 10000
N_EDGES = 320000
F = 128
NC = 2
NS = 16
L = 16

N_PAD = 10240
ROWS_PER_TILE = N_PAD // NS
CHUNK = 128


def _sc_mesh():
    return plsc.VectorSubcoreMesh(
        core_axis_name="c", subcore_axis_name="s", num_cores=NC, num_subcores=NS)


HIST_PW = 10112


def _hist_body(idx_hbm, out_hbm, idxbuf, hist):
    c = lax.axis_index("c")
    s = lax.axis_index("s")
    w = c * NS + s
    ones = jnp.ones((L,), jnp.float32)
    zeros = jnp.zeros((L,), jnp.float32)
    for a in range(4):
        row = a * (NC * NS) + w
        pltpu.sync_copy(idx_hbm.at[row], idxbuf)

        def zstep(i, _):
            hist[pl.ds(i * L, L)] = zeros
            return 0

        lax.fori_loop(0, N_PAD // L, zstep, 0)

        def step(i, _):
            iv = idxbuf[pl.ds(i * L, L)]
            plsc.addupdate_scatter(hist, [iv], ones)
            return 0

        lax.fori_loop(0, HIST_PW // L, step, 0)
        pltpu.sync_copy(hist, out_hbm.at[row])


def _histograms(idx4):
    per_w = N_EDGES // (NC * NS)
    rows = idx4.reshape(4 * NC * NS, per_w)
    rows = jnp.pad(rows, ((0, 0), (0, HIST_PW - per_w)),
                   constant_values=N_PAD - 1)
    return pl.kernel(
        _hist_body,
        out_type=jax.ShapeDtypeStruct((4 * NC * NS, N_PAD), jnp.float32),
        mesh=_sc_mesh(),
        compiler_params=pltpu.CompilerParams(needs_layout_passes=False),
        scratch_types=[
            pltpu.VMEM((HIST_PW,), jnp.int32),
            pltpu.VMEM((N_PAD,), jnp.float32),
        ],
    )(rows)


SLAB = 16


def _agg_body(n_slabs, h_hbm, sd_hbm, zeros_hbm, out_hbm,
              sd_v, rows0, rows1, agg_sh, gsem0, gsem1, ssem0, ssem1):
    c = lax.axis_index("c")
    s = lax.axis_index("s")
    pltpu.sync_copy(zeros_hbm, rows0)
    for z in range(ROWS_PER_TILE // CHUNK):
        pltpu.sync_copy(rows0, agg_sh.at[pl.ds(s * ROWS_PER_TILE + z * CHUNK, CHUNK)])
    plsc.subcore_barrier()

    def outer(t, _):
        pltpu.sync_copy(sd_hbm.at[c].at[s].at[t], sd_v)

        def inner(p, _):
            g0 = pltpu.async_copy(h_hbm.at[sd_v.at[4 * p]], rows0, gsem0)
            g1 = pltpu.async_copy(h_hbm.at[sd_v.at[4 * p + 2]], rows1, gsem1)
            g0.wait()
            s0 = pltpu.async_copy(rows0, agg_sh.at[sd_v.at[4 * p + 1]], ssem0,
                                  add=True)
            g1.wait()
            s1 = pltpu.async_copy(rows1, agg_sh.at[sd_v.at[4 * p + 3]], ssem1,
                                  add=True)
            s0.wait()
            s1.wait()
            return 0

        lax.fori_loop(0, SLAB // 2, inner, 0)
        return 0

    lax.fori_loop(0, n_slabs, outer, 0)
    plsc.subcore_barrier()
    pltpu.sync_copy(agg_sh.at[pl.ds(s * ROWS_PER_TILE, ROWS_PER_TILE)],
                    out_hbm.at[c].at[pl.ds(s * ROWS_PER_TILE, ROWS_PER_TILE)])


def _aggregate(h, sd_slab):
    n_slabs = sd_slab.shape[2]
    zeros = jnp.zeros((CHUNK, F), jnp.float32)
    return pl.kernel(
        functools.partial(_agg_body, n_slabs),
        out_type=jax.ShapeDtypeStruct((2, N_PAD, F), jnp.float32),
        mesh=_sc_mesh(),
        compiler_params=pltpu.CompilerParams(needs_layout_passes=False),
        scratch_types=[
            pltpu.VMEM((2 * SLAB, CHUNK), jnp.int32),
            pltpu.VMEM((CHUNK, F), jnp.float32),
            pltpu.VMEM((CHUNK, F), jnp.float32),
            pltpu.VMEM_SHARED((N_PAD, F), jnp.float32),
            pltpu.SemaphoreType.DMA,
            pltpu.SemaphoreType.DMA,
            pltpu.SemaphoreType.DMA,
            pltpu.SemaphoreType.DMA,
        ],
    )(h, sd_slab, zeros)


def _slab(src, dst, n_chunks):
    def prep(idx, pad_value):
        per_tile = idx.shape[1] // NS
        idx = idx.reshape(2, NS, per_tile)
        pad = n_chunks * CHUNK - per_tile
        idx = jnp.pad(idx, ((0, 0), (0, 0), (0, pad)), constant_values=pad_value)
        return idx.reshape(2, NS, n_chunks, CHUNK)

    sd = jnp.stack([prep(src, 0), prep(dst, N_NODES)], axis=3)
    return sd.reshape(2, NS, n_chunks // SLAB, 2 * SLAB, CHUNK)


def _tc_a_body(x_ref, wcat_ref, deg_ref, hcat_ref, norms_ref):
    B = deg_ref.shape[1]
    degs = deg_ref[...].reshape(4, NC * NS, B).sum(axis=1)
    norms = lax.rsqrt(jnp.maximum(degs, 1.0))
    norms_ref[...] = norms
    y = jnp.dot(x_ref[...], wcat_ref[...], preferred_element_type=jnp.float32)
    hcat_ref[0] = y[:, :F] * norms[0][:, None]
    hcat_ref[1] = y[:, F:] * norms[2][:, None]


def _tc_a(x, wcat, deg_flat):
    B = 2048
    grid = (N_PAD // B,)
    return pl.pallas_call(
        _tc_a_body,
        grid=grid,
        in_specs=[
            pl.BlockSpec((B, F), lambda i: (i, 0)),
            pl.BlockSpec((F, 2 * F), lambda i: (0, 0)),
            pl.BlockSpec((4 * NC * NS, B), lambda i: (0, i)),
        ],
        out_specs=[
            pl.BlockSpec((2, B, F), lambda i: (0, i, 0)),
            pl.BlockSpec((4, B), lambda i: (0, i)),
        ],
        out_shape=[
            jax.ShapeDtypeStruct((2, N_PAD, F), jnp.float32),
            jax.ShapeDtypeStruct((4, N_PAD), jnp.float32),
        ],
    )(x, wcat, deg_flat)


def _tc_c_body(agg_ref, norms_ref, b1i_ref, b1b_ref, w2_ref, h2_ref):
    nd_i = norms_ref[1][:, None]
    nd_b = norms_ref[3][:, None]
    ns_i = norms_ref[0][:, None]
    h = (jnp.maximum(agg_ref[0] * nd_i + b1i_ref[...], 0.0)
         + jnp.maximum(agg_ref[1] * nd_b + b1b_ref[...], 0.0))
    h2_ref[...] = jnp.dot(h, w2_ref[...], preferred_element_type=jnp.float32) * ns_i


def _tc_c(agg, norms, b1i, b1b, w2):
    B = 2048
    return pl.pallas_call(
        _tc_c_body,
        grid=(N_PAD // B,),
        in_specs=[
            pl.BlockSpec((2, B, F), lambda i: (0, i, 0)),
            pl.BlockSpec((4, B), lambda i: (0, i)),
            pl.BlockSpec((1, F), lambda i: (0, 0)),
            pl.BlockSpec((1, F), lambda i: (0, 0)),
            pl.BlockSpec((F, F), lambda i: (0, 0)),
        ],
        out_specs=pl.BlockSpec((B, F), lambda i: (i, 0)),
        out_shape=jax.ShapeDtypeStruct((N_PAD, F), jnp.float32),
    )(agg, norms, b1i, b1b, w2)


def _tc_e_body(p_ref, norms_ref, b2_ref, out_ref):
    nd_i = norms_ref[1][:, None]
    out_ref[...] = (p_ref[0] + p_ref[1]) * nd_i + b2_ref[...]


def _tc_e(p, norms, b2):
    B = 2048
    return pl.pallas_call(
        _tc_e_body,
        grid=(N_PAD // B,),
        in_specs=[
            pl.BlockSpec((2, B, F), lambda i: (0, i, 0)),
            pl.BlockSpec((4, B), lambda i: (0, i)),
            pl.BlockSpec((1, F), lambda i: (0, 0)),
        ],
        out_specs=pl.BlockSpec((B, F), lambda i: (i, 0)),
        out_shape=jax.ShapeDtypeStruct((N_PAD, F), jnp.float32),
    )(p, norms, b2)


def kernel(x, edge_index_interacts, edge_index_behave, W1i, b1i, W1b, b1b, W2, b2):
    si = edge_index_interacts[0].astype(jnp.int32)
    di = edge_index_interacts[1].astype(jnp.int32)
    sb = edge_index_behave[0].astype(jnp.int32)
    db = edge_index_behave[1].astype(jnp.int32)

    idx4 = jnp.stack([si, di, sb, db])
    deg_flat = _histograms(idx4)

    x_pad = jnp.pad(x, ((0, N_PAD - N_NODES), (0, 0)))
    wcat = jnp.concatenate([W1i, W1b], axis=1)
    hcat, norms = _tc_a(x_pad, wcat, deg_flat)
    h_gather = hcat.reshape(2 * N_PAD, F)

    n_chunks1 = -(-(N_EDGES // NS) // CHUNK)
    n_chunks1 = -(-n_chunks1 // SLAB) * SLAB
    sd1 = _slab(jnp.stack([si, sb + N_PAD]), jnp.stack([di, db]), n_chunks1)
    agg1 = _aggregate(h_gather, sd1)

    h2 = _tc_c(agg1, norms, b1i.reshape(1, F), b1b.reshape(1, F), W2)

    n_chunks2 = -(-(N_EDGES // (2 * NS)) // CHUNK)
    n_chunks2 = -(-n_chunks2 // SLAB) * SLAB
    sd2 = _slab(si.reshape(2, N_EDGES // 2), di.reshape(2, N_EDGES // 2), n_chunks2)
    p2 = _aggregate(h2, sd2)

    return _tc_e(p2, norms, b2.reshape(1, F))[:N_NODES]


# speedup vs baseline: 1.6636x; 1.6547x over previous
import functools

import jax
import jax.numpy as jnp
from jax import lax
from jax.experimental import pallas as pl
from jax.experimental.pallas import tpu as pltpu
from jax.experimental.pallas import tpu_sc as plsc

N_NODES = 10000
N_EDGES = 320000
F = 128
FH = 64
NC = 2
NS = 16
L = 16

N_PAD = 10240
ROWS_PER_TILE = N_PAD // NS
CHUNK = 128
SLAB = 8


def _sc_mesh():
    return plsc.VectorSubcoreMesh(
        core_axis_name="c", subcore_axis_name="s", num_cores=NC, num_subcores=NS)


HIST_PW = 10112


def _hist_body(idx_hbm, out_hbm, idxbuf, hist):
    c = lax.axis_index("c")
    s = lax.axis_index("s")
    w = c * NS + s
    ones = jnp.ones((L,), jnp.float32)
    zeros = jnp.zeros((L,), jnp.float32)
    for a in range(4):
        row = a * (NC * NS) + w
        pltpu.sync_copy(idx_hbm.at[row], idxbuf)

        def zstep(i, _):
            hist[pl.ds(i * L, L)] = zeros
            return 0

        lax.fori_loop(0, N_PAD // L, zstep, 0)

        def step(i, _):
            iv = idxbuf[pl.ds(i * L, L)]
            plsc.addupdate_scatter(hist, [iv], ones)
            return 0

        lax.fori_loop(0, HIST_PW // L, step, 0)
        pltpu.sync_copy(hist, out_hbm.at[row])


def _histograms(idx4):
    per_w = N_EDGES // (NC * NS)
    rows = idx4.reshape(4 * NC * NS, per_w)
    rows = jnp.pad(rows, ((0, 0), (0, HIST_PW - per_w)),
                   constant_values=N_PAD - 1)
    return pl.kernel(
        _hist_body,
        out_type=jax.ShapeDtypeStruct((4 * NC * NS, N_PAD), jnp.float32),
        mesh=_sc_mesh(),
        compiler_params=pltpu.CompilerParams(needs_layout_passes=False),
        scratch_types=[
            pltpu.VMEM((HIST_PW,), jnp.int32),
            pltpu.VMEM((N_PAD,), jnp.float32),
        ],
    )(rows)


def _agg_body(n_slabs, nh, h_hbm, sd_hbm, zeros_hbm, out_hbm,
              sd_v, rows0, rows1, sp_sh, gsem0, gsem1, ssem0, ssem1):
    c = lax.axis_index("c")
    s = lax.axis_index("s")
    for hf in range(2):
        trow = (0 if nh == 1 else c * 2) + hf
        for z in range(ROWS_PER_TILE // CHUNK):
            base = s * ROWS_PER_TILE + z * CHUNK
            pltpu.sync_copy(h_hbm.at[trow].at[pl.ds(base, CHUNK)], rows1)
            pltpu.sync_copy(rows1, sp_sh.at[pl.ds(base, CHUNK)])
        pltpu.sync_copy(zeros_hbm, rows0)
        for z in range(ROWS_PER_TILE // CHUNK):
            pltpu.sync_copy(
                rows0,
                sp_sh.at[pl.ds(N_PAD + s * ROWS_PER_TILE + z * CHUNK, CHUNK)])
        plsc.subcore_barrier()

        def outer(t, _):
            pltpu.sync_copy(sd_hbm.at[c].at[s].at[t], sd_v)

            def inner(p, _):
                g0 = pltpu.async_copy(sp_sh.at[sd_v.at[p]], rows0, gsem0)
                g0.wait()
                s0 = pltpu.async_copy(rows0, sp_sh.at[sd_v.at[SLAB + p]],
                                      ssem0, add=True)
                s0.wait()
                return 0

            lax.fori_loop(0, SLAB, inner, 0)
            return 0

        lax.fori_loop(0, n_slabs, outer, 0)
        plsc.subcore_barrier()
        pltpu.sync_copy(
            sp_sh.at[pl.ds(N_PAD + s * ROWS_PER_TILE, ROWS_PER_TILE)],
            out_hbm.at[c * 2 + hf].at[pl.ds(s * ROWS_PER_TILE, ROWS_PER_TILE)])
        plsc.subcore_barrier()


def _aggregate(hsplit, sd_slab):
    nh = hsplit.shape[0]
    hsplit = hsplit.reshape(nh * 2, N_PAD, FH)
    n_slabs = sd_slab.shape[2]
    zeros = jnp.zeros((CHUNK, FH), jnp.float32)
    out = pl.kernel(
        functools.partial(_agg_body, n_slabs, nh),
        out_type=jax.ShapeDtypeStruct((4, N_PAD, FH), jnp.float32),
        mesh=_sc_mesh(),
        compiler_params=pltpu.CompilerParams(needs_layout_passes=False,
                                             use_tc_tiling_on_sc=False),
        scratch_types=[
            pltpu.VMEM((2 * SLAB, CHUNK), jnp.int32),
            pltpu.VMEM((CHUNK, FH), jnp.float32),
            pltpu.VMEM((CHUNK, FH), jnp.float32),
            pltpu.VMEM_SHARED((2 * N_PAD, FH), jnp.float32),
            pltpu.SemaphoreType.DMA,
            pltpu.SemaphoreType.DMA,
            pltpu.SemaphoreType.DMA,
            pltpu.SemaphoreType.DMA,
        ],
    )(hsplit, sd_slab, zeros)
    return out.reshape(2, 2, N_PAD, FH)


def _slab(src, dst, n_chunks):
    def prep(idx, pad_value):
        per_tile = idx.shape[1] // NS
        idx = idx.reshape(2, NS, per_tile)
        pad = n_chunks * CHUNK - per_tile
        idx = jnp.pad(idx, ((0, 0), (0, 0), (0, pad)), constant_values=pad_value)
        return idx.reshape(2, NS, n_chunks // SLAB, SLAB, CHUNK)

    sd = jnp.stack([prep(src, 0), prep(dst + N_PAD, N_PAD + N_NODES)], axis=3)
    return sd.reshape(2, NS, n_chunks // SLAB, 2 * SLAB, CHUNK)


def _tc_a_body(x_ref, wcat_ref, deg_ref, hs_ref, norms_ref):
    B = deg_ref.shape[1]
    degs = deg_ref[...].reshape(4, NC * NS, B).sum(axis=1)
    norms = lax.rsqrt(jnp.maximum(degs, 1.0))
    norms_ref[...] = norms
    y = jnp.dot(x_ref[...], wcat_ref[...], preferred_element_type=jnp.float32)
    n0 = norms[0][:, None]
    n2 = norms[2][:, None]
    hs_ref[0, 0] = y[:, 0 * FH:1 * FH] * n0
    hs_ref[0, 1] = y[:, 1 * FH:2 * FH] * n0
    hs_ref[1, 0] = y[:, 2 * FH:3 * FH] * n2
    hs_ref[1, 1] = y[:, 3 * FH:4 * FH] * n2


def _tc_a(x, wcat, deg_flat):
    B = 2048
    return pl.pallas_call(
        _tc_a_body,
        grid=(N_PAD // B,),
        in_specs=[
            pl.BlockSpec((B, F), lambda i: (i, 0)),
            pl.BlockSpec((F, 2 * F), lambda i: (0, 0)),
            pl.BlockSpec((4 * NC * NS, B), lambda i: (0, i)),
        ],
        out_specs=[
            pl.BlockSpec((2, 2, B, FH), lambda i: (0, 0, i, 0)),
            pl.BlockSpec((4, B), lambda i: (0, i)),
        ],
        out_shape=[
            jax.ShapeDtypeStruct((2, 2, N_PAD, FH), jnp.float32),
            jax.ShapeDtypeStruct((4, N_PAD), jnp.float32),
        ],
    )(x, wcat, deg_flat)


def _tc_c_body(agg_ref, norms_ref, b1i_ref, b1b_ref, w2_ref, h2_ref):
    nd_i = norms_ref[1][:, None]
    nd_b = norms_ref[3][:, None]
    ns_i = norms_ref[0][:, None]
    agg_i = jnp.concatenate([agg_ref[0, 0], agg_ref[0, 1]], axis=-1)
    agg_b = jnp.concatenate([agg_ref[1, 0], agg_ref[1, 1]], axis=-1)
    h = (jnp.maximum(agg_i * nd_i + b1i_ref[...], 0.0)
         + jnp.maximum(agg_b * nd_b + b1b_ref[...], 0.0))
    z = jnp.dot(h, w2_ref[...], preferred_element_type=jnp.float32) * ns_i
    h2_ref[0] = z[:, :FH]
    h2_ref[1] = z[:, FH:]


def _tc_c(agg, norms, b1i, b1b, w2):
    B = 2048
    return pl.pallas_call(
        _tc_c_body,
        grid=(N_PAD // B,),
        in_specs=[
            pl.BlockSpec((2, 2, B, FH), lambda i: (0, 0, i, 0)),
            pl.BlockSpec((4, B), lambda i: (0, i)),
            pl.BlockSpec((1, F), lambda i: (0, 0)),
            pl.BlockSpec((1, F), lambda i: (0, 0)),
            pl.BlockSpec((F, F), lambda i: (0, 0)),
        ],
        out_specs=pl.BlockSpec((2, B, FH), lambda i: (0, i, 0)),
        out_shape=jax.ShapeDtypeStruct((2, N_PAD, FH), jnp.float32),
    )(agg, norms, b1i, b1b, w2)


def _tc_e_body(p_ref, norms_ref, b2_ref, out_ref):
    nd_i = norms_ref[1][:, None]
    q = p_ref[0] + p_ref[1]
    out_ref[...] = jnp.concatenate([q[0], q[1]], axis=-1) * nd_i + b2_ref[...]


def _tc_e(p, norms, b2):
    B = 2048
    return pl.pallas_call(
        _tc_e_body,
        grid=(N_PAD // B,),
        in_specs=[
            pl.BlockSpec((2, 2, B, FH), lambda i: (0, 0, i, 0)),
            pl.BlockSpec((4, B), lambda i: (0, i)),
            pl.BlockSpec((1, F), lambda i: (0, 0)),
        ],
        out_specs=pl.BlockSpec((B, F), lambda i: (i, 0)),
        out_shape=jax.ShapeDtypeStruct((N_PAD, F), jnp.float32),
    )(p, norms, b2)


def kernel(x, edge_index_interacts, edge_index_behave, W1i, b1i, W1b, b1b, W2, b2):
    si = edge_index_interacts[0].astype(jnp.int32)
    di = edge_index_interacts[1].astype(jnp.int32)
    sb = edge_index_behave[0].astype(jnp.int32)
    db = edge_index_behave[1].astype(jnp.int32)

    idx4 = jnp.stack([si, di, sb, db])
    deg_flat = _histograms(idx4)

    x_pad = jnp.pad(x, ((0, N_PAD - N_NODES), (0, 0)))
    wcat = jnp.concatenate([W1i, W1b], axis=1)
    hsplit, norms = _tc_a(x_pad, wcat, deg_flat)

    n_chunks1 = -(-(N_EDGES // NS) // CHUNK)
    n_chunks1 = -(-n_chunks1 // SLAB) * SLAB
    sd1 = _slab(jnp.stack([si, sb]), jnp.stack([di, db]), n_chunks1)
    agg1 = _aggregate(hsplit, sd1)

    h2 = _tc_c(agg1, norms, b1i.reshape(1, F), b1b.reshape(1, F), W2)

    n_chunks2 = -(-(N_EDGES // (2 * NS)) // CHUNK)
    n_chunks2 = -(-n_chunks2 // SLAB) * SLAB
    sd2 = _slab(si.reshape(2, N_EDGES // 2), di.reshape(2, N_EDGES // 2), n_chunks2)
    p2 = _aggregate(h2.reshape(1, 2, N_PAD, FH), sd2)

    return _tc_e(p2, norms, b2.reshape(1, F))[:N_NODES]


# speedup vs baseline: 1.7325x; 1.0414x over previous
import functools

import jax
import jax.numpy as jnp
from jax import lax
from jax.experimental import pallas as pl
from jax.experimental.pallas import tpu as pltpu
from jax.experimental.pallas import tpu_sc as plsc

N_NODES = 10000
N_EDGES = 320000
F = 128
FH = 64
NC = 2
NS = 16
L = 16

N_PAD = 10240
ROWS_PER_TILE = N_PAD // NS
CHUNK = 128
SLAB = 8


def _sc_mesh():
    return plsc.VectorSubcoreMesh(
        core_axis_name="c", subcore_axis_name="s", num_cores=NC, num_subcores=NS)


HIST_PW = 10112


def _hist_body(idx_hbm, out_hbm, idxbuf, hist):
    c = lax.axis_index("c")
    s = lax.axis_index("s")
    w = c * NS + s
    ones = jnp.ones((L,), jnp.float32)
    zeros = jnp.zeros((L,), jnp.float32)
    for a in range(4):
        row = a * (NC * NS) + w
        pltpu.sync_copy(idx_hbm.at[row], idxbuf)

        def zstep(i, _):
            hist[pl.ds(i * L, L)] = zeros
            return 0

        lax.fori_loop(0, N_PAD // L, zstep, 0)

        def step(i, _):
            iv = idxbuf[pl.ds(i * L, L)]
            plsc.addupdate_scatter(hist, [iv], ones)
            return 0

        lax.fori_loop(0, HIST_PW // L, step, 0)
        pltpu.sync_copy(hist, out_hbm.at[row])


def _histograms(idx4):
    per_w = N_EDGES // (NC * NS)
    rows = idx4.reshape(4 * NC * NS, per_w)
    rows = jnp.pad(rows, ((0, 0), (0, HIST_PW - per_w)),
                   constant_values=N_PAD - 1)
    return pl.kernel(
        _hist_body,
        out_type=jax.ShapeDtypeStruct((4 * NC * NS, N_PAD), jnp.float32),
        mesh=_sc_mesh(),
        compiler_params=pltpu.CompilerParams(needs_layout_passes=False),
        scratch_types=[
            pltpu.VMEM((HIST_PW,), jnp.int32),
            pltpu.VMEM((N_PAD,), jnp.float32),
        ],
    )(rows)


def _agg_body(n_slabs, nh, h_hbm, sd_hbm, zeros_hbm, out_hbm,
              sd_v, rows0, rows1, sp_sh, gsem0, gsem1, ssem0, ssem1):
    c = lax.axis_index("c")
    s = lax.axis_index("s")
    for hf in range(2):
        trow = (0 if nh == 1 else c * 2) + hf
        for z in range(ROWS_PER_TILE // CHUNK):
            base = s * ROWS_PER_TILE + z * CHUNK
            pltpu.sync_copy(h_hbm.at[trow].at[pl.ds(base, CHUNK)], rows1)
            pltpu.sync_copy(rows1, sp_sh.at[pl.ds(base, CHUNK)])
        pltpu.sync_copy(zeros_hbm, rows0)
        for z in range(ROWS_PER_TILE // CHUNK):
            pltpu.sync_copy(
                rows0,
                sp_sh.at[pl.ds(N_PAD + s * ROWS_PER_TILE + z * CHUNK, CHUNK)])
        plsc.subcore_barrier()

        def outer(t, _):
            pltpu.sync_copy(sd_hbm.at[c].at[s].at[t], sd_v)

            def inner(p, _):
                g0 = pltpu.async_copy(sp_sh.at[sd_v.at[2 * p]], rows0, gsem0)
                g1 = pltpu.async_copy(sp_sh.at[sd_v.at[2 * p + 1]], rows1,
                                      gsem1)
                g0.wait()
                s0 = pltpu.async_copy(rows0, sp_sh.at[sd_v.at[SLAB + 2 * p]],
                                      ssem0, add=True)
                g1.wait()
                s1 = pltpu.async_copy(rows1,
                                      sp_sh.at[sd_v.at[SLAB + 2 * p + 1]],
                                      ssem1, add=True)
                s0.wait()
                s1.wait()
                return 0

            lax.fori_loop(0, SLAB // 2, inner, 0)
            return 0

        lax.fori_loop(0, n_slabs, outer, 0)
        plsc.subcore_barrier()
        pltpu.sync_copy(
            sp_sh.at[pl.ds(N_PAD + s * ROWS_PER_TILE, ROWS_PER_TILE)],
            out_hbm.at[c * 2 + hf].at[pl.ds(s * ROWS_PER_TILE, ROWS_PER_TILE)])
        plsc.subcore_barrier()


def _aggregate(hsplit, sd_slab):
    nh = hsplit.shape[0]
    hsplit = hsplit.reshape(nh * 2, N_PAD, FH)
    n_slabs = sd_slab.shape[2]
    zeros = jnp.zeros((CHUNK, FH), jnp.float32)
    out = pl.kernel(
        functools.partial(_agg_body, n_slabs, nh),
        out_type=jax.ShapeDtypeStruct((4, N_PAD, FH), jnp.float32),
        mesh=_sc_mesh(),
        compiler_params=pltpu.CompilerParams(needs_layout_passes=False,
                                             use_tc_tiling_on_sc=False),
        scratch_types=[
            pltpu.VMEM((2 * SLAB, CHUNK), jnp.int32),
            pltpu.VMEM((CHUNK, FH), jnp.float32),
            pltpu.VMEM((CHUNK, FH), jnp.float32),
            pltpu.VMEM_SHARED((2 * N_PAD, FH), jnp.float32),
            pltpu.SemaphoreType.DMA,
            pltpu.SemaphoreType.DMA,
            pltpu.SemaphoreType.DMA,
            pltpu.SemaphoreType.DMA,
        ],
    )(hsplit, sd_slab, zeros)
    return out.reshape(2, 2, N_PAD, FH)


def _slab(src, dst, n_chunks):
    def prep(idx, pad_value):
        per_tile = idx.shape[1] // NS
        idx = idx.reshape(2, NS, per_tile)
        pad = n_chunks * CHUNK - per_tile
        idx = jnp.pad(idx, ((0, 0), (0, 0), (0, pad)), constant_values=pad_value)
        return idx.reshape(2, NS, n_chunks // SLAB, SLAB, CHUNK)

    sd = jnp.stack([prep(src, 0), prep(dst + N_PAD, N_PAD + N_NODES)], axis=3)
    return sd.reshape(2, NS, n_chunks // SLAB, 2 * SLAB, CHUNK)


def _tc_a_body(x_ref, wcat_ref, deg_ref, hs_ref, norms_ref):
    B = deg_ref.shape[1]
    degs = deg_ref[...].reshape(4, NC * NS, B).sum(axis=1)
    norms = lax.rsqrt(jnp.maximum(degs, 1.0))
    norms_ref[...] = norms
    y = jnp.dot(x_ref[...], wcat_ref[...], preferred_element_type=jnp.float32)
    n0 = norms[0][:, None]
    n2 = norms[2][:, None]
    hs_ref[0, 0] = y[:, 0 * FH:1 * FH] * n0
    hs_ref[0, 1] = y[:, 1 * FH:2 * FH] * n0
    hs_ref[1, 0] = y[:, 2 * FH:3 * FH] * n2
    hs_ref[1, 1] = y[:, 3 * FH:4 * FH] * n2


def _tc_a(x, wcat, deg_flat):
    B = 2048
    return pl.pallas_call(
        _tc_a_body,
        grid=(N_PAD // B,),
        in_specs=[
            pl.BlockSpec((B, F), lambda i: (i, 0)),
            pl.BlockSpec((F, 2 * F), lambda i: (0, 0)),
            pl.BlockSpec((4 * NC * NS, B), lambda i: (0, i)),
        ],
        out_specs=[
            pl.BlockSpec((2, 2, B, FH), lambda i: (0, 0, i, 0)),
            pl.BlockSpec((4, B), lambda i: (0, i)),
        ],
        out_shape=[
            jax.ShapeDtypeStruct((2, 2, N_PAD, FH), jnp.float32),
            jax.ShapeDtypeStruct((4, N_PAD), jnp.float32),
        ],
    )(x, wcat, deg_flat)


def _tc_c_body(agg_ref, norms_ref, b1i_ref, b1b_ref, w2_ref, h2_ref):
    nd_i = norms_ref[1][:, None]
    nd_b = norms_ref[3][:, None]
    ns_i = norms_ref[0][:, None]
    agg_i = jnp.concatenate([agg_ref[0, 0], agg_ref[0, 1]], axis=-1)
    agg_b = jnp.concatenate([agg_ref[1, 0], agg_ref[1, 1]], axis=-1)
    h = (jnp.maximum(agg_i * nd_i + b1i_ref[...], 0.0)
         + jnp.maximum(agg_b * nd_b + b1b_ref[...], 0.0))
    z = jnp.dot(h, w2_ref[...], preferred_element_type=jnp.float32) * ns_i
    h2_ref[0] = z[:, :FH]
    h2_ref[1] = z[:, FH:]


def _tc_c(agg, norms, b1i, b1b, w2):
    B = 2048
    return pl.pallas_call(
        _tc_c_body,
        grid=(N_PAD // B,),
        in_specs=[
            pl.BlockSpec((2, 2, B, FH), lambda i: (0, 0, i, 0)),
            pl.BlockSpec((4, B), lambda i: (0, i)),
            pl.BlockSpec((1, F), lambda i: (0, 0)),
            pl.BlockSpec((1, F), lambda i: (0, 0)),
            pl.BlockSpec((F, F), lambda i: (0, 0)),
        ],
        out_specs=pl.BlockSpec((2, B, FH), lambda i: (0, i, 0)),
        out_shape=jax.ShapeDtypeStruct((2, N_PAD, FH), jnp.float32),
    )(agg, norms, b1i, b1b, w2)


def _tc_e_body(p_ref, norms_ref, b2_ref, out_ref):
    nd_i = norms_ref[1][:, None]
    q = p_ref[0] + p_ref[1]
    out_ref[...] = jnp.concatenate([q[0], q[1]], axis=-1) * nd_i + b2_ref[...]


def _tc_e(p, norms, b2):
    B = 2048
    return pl.pallas_call(
        _tc_e_body,
        grid=(N_PAD // B,),
        in_specs=[
            pl.BlockSpec((2, 2, B, FH), lambda i: (0, 0, i, 0)),
            pl.BlockSpec((4, B), lambda i: (0, i)),
            pl.BlockSpec((1, F), lambda i: (0, 0)),
        ],
        out_specs=pl.BlockSpec((B, F), lambda i: (i, 0)),
        out_shape=jax.ShapeDtypeStruct((N_PAD, F), jnp.float32),
    )(p, norms, b2)


def kernel(x, edge_index_interacts, edge_index_behave, W1i, b1i, W1b, b1b, W2, b2):
    si = edge_index_interacts[0].astype(jnp.int32)
    di = edge_index_interacts[1].astype(jnp.int32)
    sb = edge_index_behave[0].astype(jnp.int32)
    db = edge_index_behave[1].astype(jnp.int32)

    idx4 = jnp.stack([si, di, sb, db])
    deg_flat = _histograms(idx4)

    x_pad = jnp.pad(x, ((0, N_PAD - N_NODES), (0, 0)))
    wcat = jnp.concatenate([W1i, W1b], axis=1)
    hsplit, norms = _tc_a(x_pad, wcat, deg_flat)

    n_chunks1 = -(-(N_EDGES // NS) // CHUNK)
    n_chunks1 = -(-n_chunks1 // SLAB) * SLAB
    sd1 = _slab(jnp.stack([si, sb]), jnp.stack([di, db]), n_chunks1)
    agg1 = _aggregate(hsplit, sd1)

    h2 = _tc_c(agg1, norms, b1i.reshape(1, F), b1b.reshape(1, F), W2)

    n_chunks2 = -(-(N_EDGES // (2 * NS)) // CHUNK)
    n_chunks2 = -(-n_chunks2 // SLAB) * SLAB
    sd2 = _slab(si.reshape(2, N_EDGES // 2), di.reshape(2, N_EDGES // 2), n_chunks2)
    p2 = _aggregate(h2.reshape(1, 2, N_PAD, FH), sd2)

    return _tc_e(p2, norms, b2.reshape(1, F))[:N_NODES]
